# baseline (speedup 1.0000x reference)
import functools

import jax
import jax.numpy as jnp
from jax import lax
from jax.experimental import pallas as pl
from jax.experimental.pallas import tpu as pltpu
from jax.experimental.pallas import tpu_sc as plsc

BB = 32
GG = 10000
SS = 16
LL = 4
SL = SS * LL
GAMMA = 0.001
INV_GAMMA = 1.0 / GAMMA

NC, NS = 2, 16
NW = NC * NS
GPW = 320
GPAD = NW * GPW
NGB = GPW // 16
NCH = BB // 4

_LN2 = 0.6931471805599453


def _vlog(t):
    bits = lax.bitcast_convert_type(t, jnp.int32)
    e = ((bits >> 23) - 127).astype(jnp.float32)
    m = lax.bitcast_convert_type(
        (bits & jnp.int32(0x007FFFFF)) | jnp.int32(0x3F800000), jnp.float32)
    z = (m - 1.0) / (m + 1.0)
    z2 = z * z
    logm = 2.0 * z * (1.0 + z2 * (jnp.float32(1.0 / 3.0) + z2 * jnp.float32(0.2)))
    return e * jnp.float32(_LN2) + logm


def _hi(w):
    return lax.bitcast_convert_type(w & jnp.int32(-65536), jnp.float32)


def _lo(w):
    return lax.bitcast_convert_type(w << 16, jnp.float32)


def _make_sc_call(interpret=False):
    mesh = plsc.VectorSubcoreMesh(
        core_axis_name="c", subcore_axis_name="s",
        num_cores=NC, num_subcores=NS)

    @functools.partial(
        pl.kernel,
        interpret=interpret,
        out_type=jax.ShapeDtypeStruct((BB * GPAD,), jnp.float32),
        mesh=mesh,
        compiler_params=pltpu.CompilerParams(needs_layout_passes=False),
        scratch_types=[
            pltpu.VMEM((SL * GPW,), jnp.int32),
            pltpu.VMEM((GG,), jnp.int32),
            pltpu.VMEM((GG,), jnp.int32),
            pltpu.VMEM((GPW,), jnp.float32),
            pltpu.VMEM((GPW,), jnp.float32),
            pltpu.VMEM((GPW,), jnp.float32),
            pltpu.VMEM((GPW,), jnp.float32),
        ],
    )
    def sc_clause(xp_hbm, idx_hbm, out_hbm, idx_v, xa_v, xb_v,
                  o0_v, o1_v, o2_v, o3_v):
        wid = lax.axis_index("s") * NC + lax.axis_index("c")
        pltpu.sync_copy(idx_hbm.at[pl.ds(wid * (SL * GPW), SL * GPW)], idx_v)
        orefs = (o0_v, o1_v, o2_v, o3_v)

        def ch_body(ch, carry):
            pltpu.sync_copy(xp_hbm.at[pl.ds((2 * ch) * GG, GG)], xa_v)
            pltpu.sync_copy(xp_hbm.at[pl.ds((2 * ch + 1) * GG, GG)], xb_v)

            def gb_body(gb, inner):
                col = gb * 16
                ms = [[None] * 4 for _ in range(4)]
                ts = [[None] * 4 for _ in range(4)]
                for s in range(SS):
                    j = s & 3
                    ws = []
                    for l in range(LL):
                        iv = idx_v[pl.ds((s * LL + l) * GPW + col, 16)]
                        ws.append(plsc.load_gather(xa_v, [iv]))
                        ws.append(plsc.load_gather(xb_v, [iv]))
                    for k, unp in ((0, _hi), (1, _lo), (2, _hi), (3, _lo)):
                        off = k >> 1
                        v0 = unp(ws[0 + off])
                        v1 = unp(ws[2 + off])
                        v2 = unp(ws[4 + off])
                        v3 = unp(ws[6 + off])
                        a = jnp.minimum(jnp.minimum(v0, v1),
                                        jnp.minimum(v2, v3))
                        if s < 4:
                            mn = a
                        else:
                            mn = jnp.maximum(ms[k][j], a)
                        d = (jnp.exp(mn - v0) + jnp.exp(mn - v1)
                             + jnp.exp(mn - v2) + jnp.exp(mn - v3))
                        t = 1.0 / d
                        if s < 4:
                            ts[k][j] = t
                        else:
                            ts[k][j] = ts[k][j] * jnp.exp(ms[k][j] - mn) + t
                        ms[k][j] = mn
                for k in range(4):
                    def comb(m1, t1, m2, t2):
                        m = jnp.maximum(m1, m2)
                        return m, t1 * jnp.exp(m1 - m) + t2 * jnp.exp(m2 - m)
                    m01, t01 = comb(ms[k][0], ts[k][0], ms[k][1], ts[k][1])
                    m23, t23 = comb(ms[k][2], ts[k][2], ms[k][3], ts[k][3])
                    mf, tf = comb(m01, t01, m23, t23)
                    c = (mf + _vlog(tf)) * GAMMA
                    orefs[k][pl.ds(col, 16)] = c
                return inner

            lax.fori_loop(0, NGB, gb_body, 0)
            for k, brow in enumerate((2 * ch, 2 * ch + 16,
                                      2 * ch + 1, 2 * ch + 17)):
                pltpu.sync_copy(
                    orefs[k], out_hbm.at[pl.ds(brow * GPAD + wid * GPW, GPW)])
            return carry

        lax.fori_loop(0, NCH, ch_body, 0)

    return sc_clause


_SC_CALL_CACHE = []


def kernel(x, I_i):
    if not _SC_CALL_CACHE:
        _SC_CALL_CACHE.append(_make_sc_call())
    sc_clause = _SC_CALL_CACHE[0]
    y16 = (x * INV_GAMMA).astype(jnp.bfloat16)
    hi = lax.bitcast_convert_type(y16[:16], jnp.uint16).astype(jnp.uint32) << 16
    lo = lax.bitcast_convert_type(y16[16:], jnp.uint16).astype(jnp.uint32)
    xp = lax.bitcast_convert_type(hi | lo, jnp.int32)
    idx = I_i.reshape(GG, SL).astype(jnp.int32)
    idx = jnp.pad(idx, ((0, GPAD - GG), (0, 0)))
    idx = idx.reshape(NW, GPW, SL).transpose(0, 2, 1).reshape(-1)
    out = sc_clause(xp.reshape(-1), idx)
    return out.reshape(BB, GPAD)[:, :GG]


# speedup vs baseline: 1.4107x; 1.4107x over previous
import functools

import jax
import jax.numpy as jnp
import numpy as np
from jax import lax
from jax.experimental import pallas as pl
from jax.experimental.pallas import tpu as pltpu
from jax.experimental.pallas import tpu_sc as plsc

BB = 32
GG = 10000
SS = 16
LL = 4
SL = SS * LL
GAMMA = 0.001
INV_GAMMA = 1.0 / GAMMA

NC, NS = 2, 16
NW = NC * NS
GPW = 320
GPAD = NW * GPW
NGB = GPW // 16
NCH = BB // 4

NTAB = 17536


def _softplus_table() -> np.ndarray:
    bits = (np.arange(NTAB, dtype=np.uint32) << 16).view(np.float32)
    return np.log1p(np.exp(-bits.astype(np.float64))).astype(np.float32)


_TAB = _softplus_table()


def _hi(w):
    return lax.bitcast_convert_type(w & jnp.int32(-65536), jnp.float32)


def _lo(w):
    return lax.bitcast_convert_type(w << 16, jnp.float32)


def _gtab(tab_v, d):
    bits = lax.bitcast_convert_type(d, jnp.int32)
    return plsc.load_gather(tab_v, [lax.shift_right_logical(bits, 16)])


def _sa(tab_v, a, b):
    return jnp.minimum(a, b) - _gtab(tab_v, jnp.abs(a - b))


def _so(tab_v, a, b):
    return jnp.maximum(a, b) + _gtab(tab_v, jnp.abs(a - b))


def _make_sc_call(interpret=False):
    mesh = plsc.VectorSubcoreMesh(
        core_axis_name="c", subcore_axis_name="s",
        num_cores=NC, num_subcores=NS)

    @functools.partial(
        pl.kernel,
        interpret=interpret,
        out_type=jax.ShapeDtypeStruct((BB * GPAD,), jnp.float32),
        mesh=mesh,
        compiler_params=pltpu.CompilerParams(needs_layout_passes=False),
        scratch_types=[
            pltpu.VMEM((SL * GPW,), jnp.int32),
            pltpu.VMEM((NTAB,), jnp.float32),
            pltpu.VMEM((GG,), jnp.int32),
            pltpu.VMEM((GG,), jnp.int32),
            pltpu.VMEM((GPW,), jnp.float32),
            pltpu.VMEM((GPW,), jnp.float32),
            pltpu.VMEM((GPW,), jnp.float32),
            pltpu.VMEM((GPW,), jnp.float32),
        ],
    )
    def sc_clause(xp_hbm, idx_hbm, tab_hbm, out_hbm, idx_v, tab_v,
                  xa_v, xb_v, o0_v, o1_v, o2_v, o3_v):
        wid = lax.axis_index("s") * NC + lax.axis_index("c")
        pltpu.sync_copy(idx_hbm.at[pl.ds(wid * (SL * GPW), SL * GPW)], idx_v)
        pltpu.sync_copy(tab_hbm, tab_v)
        orefs = (o0_v, o1_v, o2_v, o3_v)

        def ch_body(ch, carry):
            pltpu.sync_copy(xp_hbm.at[pl.ds((2 * ch) * GG, GG)], xa_v)
            pltpu.sync_copy(xp_hbm.at[pl.ds((2 * ch + 1) * GG, GG)], xb_v)

            def gb_body(gb, inner):
                col = gb * 16
                stacks = [[] for _ in range(4)]
                for s in range(SS):
                    ws = []
                    for l in range(LL):
                        iv = idx_v[pl.ds((s * LL + l) * GPW + col, 16)]
                        ws.append(plsc.load_gather(xa_v, [iv]))
                        ws.append(plsc.load_gather(xb_v, [iv]))
                    for k, unp in ((0, _hi), (1, _lo), (2, _hi), (3, _lo)):
                        off = k >> 1
                        v0 = unp(ws[0 + off])
                        v1 = unp(ws[2 + off])
                        v2 = unp(ws[4 + off])
                        v3 = unp(ws[6 + off])
                        p = _sa(tab_v, _sa(tab_v, v0, v1), _sa(tab_v, v2, v3))
                        cnt = s + 1
                        while cnt % 2 == 0:
                            p = _so(tab_v, stacks[k].pop(), p)
                            cnt //= 2
                        stacks[k].append(p)
                for k in range(4):
                    orefs[k][pl.ds(col, 16)] = stacks[k][0] * GAMMA
                return inner

            lax.fori_loop(0, NGB, gb_body, 0)
            for k, brow in enumerate((2 * ch, 2 * ch + 16,
                                      2 * ch + 1, 2 * ch + 17)):
                pltpu.sync_copy(
                    orefs[k], out_hbm.at[pl.ds(brow * GPAD + wid * GPW, GPW)])
            return carry

        lax.fori_loop(0, NCH, ch_body, 0)

    return sc_clause


_SC_CALL_CACHE = []


def kernel(x, I_i):
    if not _SC_CALL_CACHE:
        _SC_CALL_CACHE.append(_make_sc_call())
    sc_clause = _SC_CALL_CACHE[0]
    y16 = (x * INV_GAMMA).astype(jnp.bfloat16)
    hi = lax.bitcast_convert_type(y16[:16], jnp.uint16).astype(jnp.uint32) << 16
    lo = lax.bitcast_convert_type(y16[16:], jnp.uint16).astype(jnp.uint32)
    xp = lax.bitcast_convert_type(hi | lo, jnp.int32)
    idx = I_i.reshape(GG, SL).astype(jnp.int32)
    idx = jnp.pad(idx, ((0, GPAD - GG), (0, 0)))
    idx = idx.reshape(NW, GPW, SL).transpose(0, 2, 1).reshape(-1)
    out = sc_clause(xp.reshape(-1), idx, jnp.asarray(_TAB))
    return out.reshape(BB, GPAD)[:, :GG]
